# baseline (speedup 1.0000x reference)
import functools

import jax
import jax.numpy as jnp
from jax.experimental import pallas as pl
from jax.experimental.pallas import tpu as pltpu

EPS = 1e-6


def _moe_kernel(xp_ref, lnpre_ref, rw_ref, sg_ref, su_ref, sd_ref,
                eg_ref, eu_ref, ed_ref, ls_ref, lnpost_ref,
                out_ref, fn_ref, i1_ref, i2_ref, w1_ref, acc_ref,
                egb_ref, eub_ref, edb_ref, sgb_ref, sub_ref, sdb_ref,
                e_sem, sh_sem, *, n_experts):
    e = pl.program_id(0)
    hid = eg_ref.shape[2]
    h2 = hid // 2

    def _chunk_copies(idx, slot, half):
        lo, hi = half * h2, (half + 1) * h2
        return [
            pltpu.make_async_copy(eg_ref.at[idx, :, lo:hi],
                                  egb_ref.at[slot, :, lo:hi],
                                  e_sem.at[slot, half, 0]),
            pltpu.make_async_copy(eu_ref.at[idx, :, lo:hi],
                                  eub_ref.at[slot, :, lo:hi],
                                  e_sem.at[slot, half, 1]),
            pltpu.make_async_copy(ed_ref.at[idx, lo:hi, :],
                                  edb_ref.at[slot, lo:hi, :],
                                  e_sem.at[slot, half, 2]),
        ]

    def start_expert(idx, slot):
        for half in (0, 1):
            for c in _chunk_copies(idx, slot, half):
                c.start()

    @pl.when(e == 0)
    def _prologue():
        start_expert(0, 0)
        start_expert(1, 1)
        pltpu.make_async_copy(sg_ref, sgb_ref, sh_sem.at[0]).start()
        pltpu.make_async_copy(su_ref, sub_ref, sh_sem.at[1]).start()
        pltpu.make_async_copy(sd_ref, sdb_ref, sh_sem.at[2]).start()
        h = xp_ref[...]
        var = jnp.mean(h * h, axis=-1, keepdims=True)
        fn = (h * jax.lax.rsqrt(var + EPS)) * lnpre_ref[...]
        fn_ref[...] = fn.astype(jnp.bfloat16)
        logits = jnp.dot(fn, rw_ref[...], preferred_element_type=jnp.float32)
        n, ne = logits.shape
        iota = jax.lax.broadcasted_iota(jnp.int32, (n, ne), 1)
        m1 = jnp.max(logits, axis=-1, keepdims=True)
        i1 = jnp.min(jnp.where(logits == m1, iota, ne), axis=-1, keepdims=True)
        masked = jnp.where(iota == i1, -jnp.inf, logits)
        m2 = jnp.max(masked, axis=-1, keepdims=True)
        i2 = jnp.min(jnp.where(masked == m2, iota, ne), axis=-1, keepdims=True)
        i1_ref[...] = i1
        i2_ref[...] = i2
        w1_ref[...] = jax.nn.sigmoid(m1 - m2)

    slot = jax.lax.rem(e, 2)
    fnb = fn_ref[...]
    w1 = w1_ref[...]
    ce = (jnp.where(i1_ref[...] == e, w1, 0.0)
          + jnp.where(i2_ref[...] == e, 1.0 - w1, 0.0)).astype(jnp.bfloat16)

    for half in (0, 1):
        lo, hi = half * h2, (half + 1) * h2
        for c in _chunk_copies(e, slot, half):
            c.wait()
        g = jnp.dot(fnb, egb_ref[slot, :, lo:hi].astype(jnp.bfloat16),
                    preferred_element_type=jnp.float32)
        u = jnp.dot(fnb, eub_ref[slot, :, lo:hi].astype(jnp.bfloat16),
                    preferred_element_type=jnp.float32)
        hmid = (jax.nn.silu(g) * u).astype(jnp.bfloat16) * ce
        contrib = jnp.dot(hmid, edb_ref[slot, lo:hi, :].astype(jnp.bfloat16),
                          preferred_element_type=jnp.float32)
        if half == 0:
            @pl.when(e == 0)
            def _init():
                acc_ref[...] = contrib

            @pl.when(e != 0)
            def _accum():
                acc_ref[...] += contrib
        else:
            acc_ref[...] += contrib

    @pl.when(e + 2 < n_experts)
    def _prefetch_next():
        start_expert(e + 2, slot)

    @pl.when(e == 3)
    def _shared():
        pltpu.make_async_copy(sg_ref, sgb_ref, sh_sem.at[0]).wait()
        pltpu.make_async_copy(su_ref, sub_ref, sh_sem.at[1]).wait()
        pltpu.make_async_copy(sd_ref, sdb_ref, sh_sem.at[2]).wait()
        g = jnp.dot(fn_ref[...], sgb_ref[...].astype(jnp.bfloat16),
                    preferred_element_type=jnp.float32)
        u = jnp.dot(fn_ref[...], sub_ref[...].astype(jnp.bfloat16),
                    preferred_element_type=jnp.float32)
        acc_ref[...] += jnp.dot((jax.nn.silu(g) * u).astype(jnp.bfloat16),
                                sdb_ref[...].astype(jnp.bfloat16),
                                preferred_element_type=jnp.float32)

    @pl.when(e == n_experts - 1)
    def _epilogue():
        a = acc_ref[...] * ls_ref[...]
        var = jnp.mean(a * a, axis=-1, keepdims=True)
        out_ref[...] = (a * jax.lax.rsqrt(var + EPS)) * lnpost_ref[...]


@jax.jit
def kernel(x, ln_pre_w, router_w, sh_gate, sh_up, sh_down, eg, eu, ed,
           layer_scale, ln_post_w):
    b, t, d = x.shape
    in_dim = ln_pre_w.shape[0]
    k_pool = in_dim // d
    t2 = (t // k_pool) * k_pool
    n = b * (t2 // k_pool)
    n_experts = router_w.shape[1]
    hid = sh_gate.shape[1]
    out_dim = sh_down.shape[1]

    xp = x[:, :t2, :].reshape(n, in_dim)
    full = lambda shape: pl.BlockSpec(shape, lambda e: (0,) * len(shape))
    anyspec = pl.BlockSpec(memory_space=pl.ANY)

    out = pl.pallas_call(
        functools.partial(_moe_kernel, n_experts=n_experts),
        grid=(n_experts,),
        in_specs=[
            full((n, in_dim)),
            full((1, in_dim)),
            full((in_dim, n_experts)),
            anyspec,
            anyspec,
            anyspec,
            anyspec,
            anyspec,
            anyspec,
            full((1, out_dim)),
            full((1, out_dim)),
        ],
        out_specs=full((n, out_dim)),
        out_shape=jax.ShapeDtypeStruct((n, out_dim), jnp.float32),
        scratch_shapes=[
            pltpu.VMEM((n, in_dim), jnp.bfloat16),
            pltpu.VMEM((n, 1), jnp.int32),
            pltpu.VMEM((n, 1), jnp.int32),
            pltpu.VMEM((n, 1), jnp.float32),
            pltpu.VMEM((n, out_dim), jnp.float32),
            pltpu.VMEM((2, in_dim, hid), jnp.float32),
            pltpu.VMEM((2, in_dim, hid), jnp.float32),
            pltpu.VMEM((2, hid, out_dim), jnp.float32),
            pltpu.VMEM((in_dim, hid), jnp.float32),
            pltpu.VMEM((in_dim, hid), jnp.float32),
            pltpu.VMEM((hid, out_dim), jnp.float32),
            pltpu.SemaphoreType.DMA((2, 2, 3)),
            pltpu.SemaphoreType.DMA((3,)),
        ],
    )(xp, ln_pre_w.reshape(1, in_dim), router_w, sh_gate, sh_up, sh_down,
      eg, eu, ed, layer_scale.reshape(1, out_dim), ln_post_w.reshape(1, out_dim))
    return out.reshape(b, t2 // k_pool, out_dim)


# speedup vs baseline: 1.0068x; 1.0068x over previous
import functools

import jax
import jax.numpy as jnp
from jax.experimental import pallas as pl
from jax.experimental.pallas import tpu as pltpu

EPS = 1e-6


def _moe_kernel(xp_ref, lnpre_ref, rw_ref, sg_ref, su_ref, sd_ref,
                eg_ref, eu_ref, ed_ref, ls_ref, lnpost_ref,
                out_ref, fn_ref, i1_ref, i2_ref, w1_ref, acc_ref,
                egb_ref, eub_ref, edb_ref, sgb_ref, sub_ref, sdb_ref,
                sg_sem, su_sem, sd_sem, sh_sem, *, n_experts):
    e = pl.program_id(0)

    def _copies(idx, slot):
        h2 = eg_ref.shape[2] // 2
        i2h = ed_ref.shape[1]
        return [
            pltpu.make_async_copy(eg_ref.at[idx, :, :h2],
                                  egb_ref.at[slot, :, :h2], sg_sem.at[slot, 0]),
            pltpu.make_async_copy(eg_ref.at[idx, :, h2:],
                                  egb_ref.at[slot, :, h2:], sg_sem.at[slot, 1]),
            pltpu.make_async_copy(eu_ref.at[idx, :, :h2],
                                  eub_ref.at[slot, :, :h2], su_sem.at[slot, 0]),
            pltpu.make_async_copy(eu_ref.at[idx, :, h2:],
                                  eub_ref.at[slot, :, h2:], su_sem.at[slot, 1]),
            pltpu.make_async_copy(ed_ref.at[idx, :i2h // 2],
                                  edb_ref.at[slot, :i2h // 2], sd_sem.at[slot, 0]),
            pltpu.make_async_copy(ed_ref.at[idx, i2h // 2:],
                                  edb_ref.at[slot, i2h // 2:], sd_sem.at[slot, 1]),
        ]

    def start(idx, slot):
        for c in _copies(idx, slot):
            c.start()

    @pl.when(e == 0)
    def _prologue():
        start(0, 0)
        start(1, 1)
        pltpu.make_async_copy(sg_ref, sgb_ref, sh_sem.at[0]).start()
        pltpu.make_async_copy(su_ref, sub_ref, sh_sem.at[1]).start()
        pltpu.make_async_copy(sd_ref, sdb_ref, sh_sem.at[2]).start()
        h = xp_ref[...]
        var = jnp.mean(h * h, axis=-1, keepdims=True)
        fn = (h * jax.lax.rsqrt(var + EPS)) * lnpre_ref[...]
        fn_ref[...] = fn.astype(jnp.bfloat16)
        logits = jnp.dot(fn, rw_ref[...], preferred_element_type=jnp.float32)
        n, ne = logits.shape
        iota = jax.lax.broadcasted_iota(jnp.int32, (n, ne), 1)
        m1 = jnp.max(logits, axis=-1, keepdims=True)
        i1 = jnp.min(jnp.where(logits == m1, iota, ne), axis=-1, keepdims=True)
        masked = jnp.where(iota == i1, -jnp.inf, logits)
        m2 = jnp.max(masked, axis=-1, keepdims=True)
        i2 = jnp.min(jnp.where(masked == m2, iota, ne), axis=-1, keepdims=True)
        i1_ref[...] = i1
        i2_ref[...] = i2
        w1_ref[...] = jax.nn.sigmoid(m1 - m2)

    slot = jax.lax.rem(e, 2)
    for c in _copies(e, slot):
        c.wait()

    fnb = fn_ref[...]
    g = jnp.dot(fnb, egb_ref[slot].astype(jnp.bfloat16),
                preferred_element_type=jnp.float32)
    u = jnp.dot(fnb, eub_ref[slot].astype(jnp.bfloat16),
                preferred_element_type=jnp.float32)
    hmid = jax.nn.silu(g) * u
    w1 = w1_ref[...]
    ce = (jnp.where(i1_ref[...] == e, w1, 0.0)
          + jnp.where(i2_ref[...] == e, 1.0 - w1, 0.0))
    contrib = jnp.dot((hmid * ce).astype(jnp.bfloat16),
                      edb_ref[slot].astype(jnp.bfloat16),
                      preferred_element_type=jnp.float32)

    @pl.when(e == 0)
    def _init_acc():
        acc_ref[...] = contrib

    @pl.when(e != 0)
    def _add_acc():
        acc_ref[...] += contrib

    @pl.when(e + 2 < n_experts)
    def _prefetch_next():
        start(e + 2, slot)

    @pl.when(e == 3)
    def _shared():
        pltpu.make_async_copy(sg_ref, sgb_ref, sh_sem.at[0]).wait()
        pltpu.make_async_copy(su_ref, sub_ref, sh_sem.at[1]).wait()
        pltpu.make_async_copy(sd_ref, sdb_ref, sh_sem.at[2]).wait()
        fb = fn_ref[...]
        sg = jnp.dot(fb, sgb_ref[...].astype(jnp.bfloat16),
                     preferred_element_type=jnp.float32)
        su = jnp.dot(fb, sub_ref[...].astype(jnp.bfloat16),
                     preferred_element_type=jnp.float32)
        acc_ref[...] += jnp.dot((jax.nn.silu(sg) * su).astype(jnp.bfloat16),
                                sdb_ref[...].astype(jnp.bfloat16),
                                preferred_element_type=jnp.float32)

    @pl.when(e == n_experts - 1)
    def _epilogue():
        a = acc_ref[...] * ls_ref[...]
        var = jnp.mean(a * a, axis=-1, keepdims=True)
        out_ref[...] = (a * jax.lax.rsqrt(var + EPS)) * lnpost_ref[...]


@jax.jit
def kernel(x, ln_pre_w, router_w, sh_gate, sh_up, sh_down, eg, eu, ed,
           layer_scale, ln_post_w):
    b, t, d = x.shape
    in_dim = ln_pre_w.shape[0]
    k_pool = in_dim // d
    t2 = (t // k_pool) * k_pool
    n = b * (t2 // k_pool)
    n_experts = router_w.shape[1]
    hid = sh_gate.shape[1]
    out_dim = sh_down.shape[1]

    xp = x[:, :t2, :].reshape(n, in_dim)
    full = lambda shape: pl.BlockSpec(shape, lambda e: (0,) * len(shape))
    anyspec = pl.BlockSpec(memory_space=pl.ANY)

    out = pl.pallas_call(
        functools.partial(_moe_kernel, n_experts=n_experts),
        grid=(n_experts,),
        in_specs=[
            full((n, in_dim)),
            full((1, in_dim)),
            full((in_dim, n_experts)),
            anyspec,
            anyspec,
            anyspec,
            anyspec,
            anyspec,
            anyspec,
            full((1, out_dim)),
            full((1, out_dim)),
        ],
        out_specs=full((n, out_dim)),
        out_shape=jax.ShapeDtypeStruct((n, out_dim), jnp.float32),
        scratch_shapes=[
            pltpu.VMEM((n, in_dim), jnp.bfloat16),
            pltpu.VMEM((n, 1), jnp.int32),
            pltpu.VMEM((n, 1), jnp.int32),
            pltpu.VMEM((n, 1), jnp.float32),
            pltpu.VMEM((n, out_dim), jnp.float32),
            pltpu.VMEM((2, in_dim, hid), jnp.float32),
            pltpu.VMEM((2, in_dim, hid), jnp.float32),
            pltpu.VMEM((2, hid, out_dim), jnp.float32),
            pltpu.VMEM((in_dim, hid), jnp.float32),
            pltpu.VMEM((in_dim, hid), jnp.float32),
            pltpu.VMEM((hid, out_dim), jnp.float32),
            pltpu.SemaphoreType.DMA((2, 2)),
            pltpu.SemaphoreType.DMA((2, 2)),
            pltpu.SemaphoreType.DMA((2, 2)),
            pltpu.SemaphoreType.DMA((3,)),
        ],
    )(xp, ln_pre_w.reshape(1, in_dim), router_w, sh_gate, sh_up, sh_down,
      eg, eu, ed, layer_scale.reshape(1, out_dim), ln_post_w.reshape(1, out_dim))
    return out.reshape(b, t2 // k_pool, out_dim)


# speedup vs baseline: 1.0995x; 1.0920x over previous
import functools

import jax
import jax.numpy as jnp
from jax.experimental import pallas as pl
from jax.experimental.pallas import tpu as pltpu

EPS = 1e-6


def _moe_kernel(xp_ref, lnpre_ref, rw_ref, sg_ref, su_ref, sd_ref,
                eg_ref, eu_ref, ed_ref, ls_ref, lnpost_ref,
                out_ref, fn_ref, i1_ref, i2_ref, w1_ref, acc_ref,
                egb_ref, eub_ref, edb_ref, sg_sem, su_sem, sd_sem,
                *, n_experts):
    e = pl.program_id(0)

    def _copies(idx, slot):
        h2 = eg_ref.shape[2] // 2
        i2h = ed_ref.shape[1]
        return [
            pltpu.make_async_copy(eg_ref.at[idx, :, :h2],
                                  egb_ref.at[slot, :, :h2], sg_sem.at[slot, 0]),
            pltpu.make_async_copy(eg_ref.at[idx, :, h2:],
                                  egb_ref.at[slot, :, h2:], sg_sem.at[slot, 1]),
            pltpu.make_async_copy(eu_ref.at[idx, :, :h2],
                                  eub_ref.at[slot, :, :h2], su_sem.at[slot, 0]),
            pltpu.make_async_copy(eu_ref.at[idx, :, h2:],
                                  eub_ref.at[slot, :, h2:], su_sem.at[slot, 1]),
            pltpu.make_async_copy(ed_ref.at[idx, :i2h // 2],
                                  edb_ref.at[slot, :i2h // 2], sd_sem.at[slot, 0]),
            pltpu.make_async_copy(ed_ref.at[idx, i2h // 2:],
                                  edb_ref.at[slot, i2h // 2:], sd_sem.at[slot, 1]),
        ]

    def start(idx, slot):
        for c in _copies(idx, slot):
            c.start()

    @pl.when(e == 0)
    def _prologue():
        start(0, 0)
        start(1, 1)
        start(2, 2)
        h = xp_ref[...]
        var = jnp.mean(h * h, axis=-1, keepdims=True)
        fn = (h * jax.lax.rsqrt(var + EPS)) * lnpre_ref[...]
        fn_ref[...] = fn.astype(jnp.bfloat16)
        logits = jnp.dot(fn, rw_ref[...], preferred_element_type=jnp.float32)
        n, ne = logits.shape
        iota = jax.lax.broadcasted_iota(jnp.int32, (n, ne), 1)
        m1 = jnp.max(logits, axis=-1, keepdims=True)
        i1 = jnp.min(jnp.where(logits == m1, iota, ne), axis=-1, keepdims=True)
        masked = jnp.where(iota == i1, -jnp.inf, logits)
        m2 = jnp.max(masked, axis=-1, keepdims=True)
        i2 = jnp.min(jnp.where(masked == m2, iota, ne), axis=-1, keepdims=True)
        i1_ref[...] = i1
        i2_ref[...] = i2
        w1_ref[...] = jax.nn.sigmoid(m1 - m2)
        fnb = fn_ref[...]
        g = jnp.dot(fnb, sg_ref[...].astype(jnp.bfloat16),
                    preferred_element_type=jnp.float32)
        u = jnp.dot(fnb, su_ref[...].astype(jnp.bfloat16),
                    preferred_element_type=jnp.float32)
        acc_ref[...] = jnp.dot((jax.nn.silu(g) * u).astype(jnp.bfloat16),
                               sd_ref[...].astype(jnp.bfloat16),
                               preferred_element_type=jnp.float32)

    slot = jax.lax.rem(e, 3)
    for c in _copies(e, slot):
        c.wait()

    fnb = fn_ref[...]
    g = jnp.dot(fnb, egb_ref[slot].astype(jnp.bfloat16),
                preferred_element_type=jnp.float32)
    u = jnp.dot(fnb, eub_ref[slot].astype(jnp.bfloat16),
                preferred_element_type=jnp.float32)
    hmid = jax.nn.silu(g) * u
    w1 = w1_ref[...]
    ce = (jnp.where(i1_ref[...] == e, w1, 0.0)
          + jnp.where(i2_ref[...] == e, 1.0 - w1, 0.0))
    acc_ref[...] += jnp.dot((hmid * ce).astype(jnp.bfloat16),
                            edb_ref[slot].astype(jnp.bfloat16),
                            preferred_element_type=jnp.float32)

    @pl.when(e + 3 < n_experts)
    def _prefetch_next():
        start(e + 3, slot)

    @pl.when(e == n_experts - 1)
    def _epilogue():
        a = acc_ref[...] * ls_ref[...]
        var = jnp.mean(a * a, axis=-1, keepdims=True)
        out_ref[...] = (a * jax.lax.rsqrt(var + EPS)) * lnpost_ref[...]


@jax.jit
def kernel(x, ln_pre_w, router_w, sh_gate, sh_up, sh_down, eg, eu, ed,
           layer_scale, ln_post_w):
    b, t, d = x.shape
    in_dim = ln_pre_w.shape[0]
    k_pool = in_dim // d
    t2 = (t // k_pool) * k_pool
    n = b * (t2 // k_pool)
    n_experts = router_w.shape[1]
    hid = sh_gate.shape[1]
    out_dim = sh_down.shape[1]

    xp = x[:, :t2, :].reshape(n, in_dim)
    full = lambda shape: pl.BlockSpec(shape, lambda e: (0,) * len(shape))
    anyspec = pl.BlockSpec(memory_space=pl.ANY)

    out = pl.pallas_call(
        functools.partial(_moe_kernel, n_experts=n_experts),
        grid=(n_experts,),
        in_specs=[
            full((n, in_dim)),
            full((1, in_dim)),
            full((in_dim, n_experts)),
            full((in_dim, hid)),
            full((in_dim, hid)),
            full((hid, out_dim)),
            anyspec,
            anyspec,
            anyspec,
            full((1, out_dim)),
            full((1, out_dim)),
        ],
        out_specs=full((n, out_dim)),
        out_shape=jax.ShapeDtypeStruct((n, out_dim), jnp.float32),
        scratch_shapes=[
            pltpu.VMEM((n, in_dim), jnp.bfloat16),
            pltpu.VMEM((n, 1), jnp.int32),
            pltpu.VMEM((n, 1), jnp.int32),
            pltpu.VMEM((n, 1), jnp.float32),
            pltpu.VMEM((n, out_dim), jnp.float32),
            pltpu.VMEM((3, in_dim, hid), jnp.float32),
            pltpu.VMEM((3, in_dim, hid), jnp.float32),
            pltpu.VMEM((3, hid, out_dim), jnp.float32),
            pltpu.SemaphoreType.DMA((3, 2)),
            pltpu.SemaphoreType.DMA((3, 2)),
            pltpu.SemaphoreType.DMA((3, 2)),
        ],
    )(xp, ln_pre_w.reshape(1, in_dim), router_w, sh_gate, sh_up, sh_down,
      eg, eu, ed, layer_scale.reshape(1, out_dim), ln_post_w.reshape(1, out_dim))
    return out.reshape(b, t2 // k_pool, out_dim)


# speedup vs baseline: 1.1037x; 1.0038x over previous
import functools

import jax
import jax.numpy as jnp
from jax.experimental import pallas as pl
from jax.experimental.pallas import tpu as pltpu

EPS = 1e-6


def _moe_kernel(xp_ref, lnpre_ref, rw_ref, sg_ref, su_ref, sd_ref,
                eg_ref, eu_ref, ed_ref, ls_ref, lnpost_ref,
                out_ref, fn_ref, i1_ref, i2_ref, w1_ref, acc_ref,
                egb_ref, eub_ref, edb_ref, sg_sem, su_sem, sd_sem,
                *, n_experts):
    e = pl.program_id(0)

    def _copies(idx, slot):
        h2 = eg_ref.shape[2] // 2
        i2h = ed_ref.shape[1]
        return [
            pltpu.make_async_copy(eg_ref.at[idx, :, :h2],
                                  egb_ref.at[slot, :, :h2], sg_sem.at[slot, 0]),
            pltpu.make_async_copy(eg_ref.at[idx, :, h2:],
                                  egb_ref.at[slot, :, h2:], sg_sem.at[slot, 1]),
            pltpu.make_async_copy(eu_ref.at[idx, :, :h2],
                                  eub_ref.at[slot, :, :h2], su_sem.at[slot, 0]),
            pltpu.make_async_copy(eu_ref.at[idx, :, h2:],
                                  eub_ref.at[slot, :, h2:], su_sem.at[slot, 1]),
            pltpu.make_async_copy(ed_ref.at[idx, :i2h // 2],
                                  edb_ref.at[slot, :i2h // 2], sd_sem.at[slot, 0]),
            pltpu.make_async_copy(ed_ref.at[idx, i2h // 2:],
                                  edb_ref.at[slot, i2h // 2:], sd_sem.at[slot, 1]),
        ]

    def start(idx, slot):
        for c in _copies(idx, slot):
            c.start()

    @pl.when(e == 0)
    def _prologue():
        start(0, 0)
        start(1, 1)
        h = xp_ref[...]
        var = jnp.mean(h * h, axis=-1, keepdims=True)
        fn = (h * jax.lax.rsqrt(var + EPS)) * lnpre_ref[...]
        fn_ref[...] = fn.astype(jnp.bfloat16)
        logits = jnp.dot(fn, rw_ref[...], preferred_element_type=jnp.float32)
        n, ne = logits.shape
        iota = jax.lax.broadcasted_iota(jnp.int32, (n, ne), 1)
        m1 = jnp.max(logits, axis=-1, keepdims=True)
        i1 = jnp.min(jnp.where(logits == m1, iota, ne), axis=-1, keepdims=True)
        masked = jnp.where(iota == i1, -jnp.inf, logits)
        m2 = jnp.max(masked, axis=-1, keepdims=True)
        i2 = jnp.min(jnp.where(masked == m2, iota, ne), axis=-1, keepdims=True)
        i1_ref[...] = i1
        i2_ref[...] = i2
        w1_ref[...] = jax.nn.sigmoid(m1 - m2)
        fnb = fn_ref[...]
        g = jnp.dot(fnb, sg_ref[...].astype(jnp.bfloat16),
                    preferred_element_type=jnp.float32)
        u = jnp.dot(fnb, su_ref[...].astype(jnp.bfloat16),
                    preferred_element_type=jnp.float32)
        acc_ref[...] = jnp.dot((jax.nn.silu(g) * u).astype(jnp.bfloat16),
                               sd_ref[...].astype(jnp.bfloat16),
                               preferred_element_type=jnp.float32)

    slot = jax.lax.rem(e, 2)
    for c in _copies(e, slot):
        c.wait()

    fnb = fn_ref[...]
    g = jnp.dot(fnb, egb_ref[slot].astype(jnp.bfloat16),
                preferred_element_type=jnp.float32)
    u = jnp.dot(fnb, eub_ref[slot].astype(jnp.bfloat16),
                preferred_element_type=jnp.float32)
    hmid = jax.nn.silu(g) * u
    w1 = w1_ref[...]
    ce = (jnp.where(i1_ref[...] == e, w1, 0.0)
          + jnp.where(i2_ref[...] == e, 1.0 - w1, 0.0))
    acc_ref[...] += jnp.dot((hmid * ce).astype(jnp.bfloat16),
                            edb_ref[slot].astype(jnp.bfloat16),
                            preferred_element_type=jnp.float32)

    @pl.when(e + 2 < n_experts)
    def _prefetch_next():
        start(e + 2, slot)

    @pl.when(e == n_experts - 1)
    def _epilogue():
        a = acc_ref[...] * ls_ref[...]
        var = jnp.mean(a * a, axis=-1, keepdims=True)
        out_ref[...] = (a * jax.lax.rsqrt(var + EPS)) * lnpost_ref[...]


@jax.jit
def kernel(x, ln_pre_w, router_w, sh_gate, sh_up, sh_down, eg, eu, ed,
           layer_scale, ln_post_w):
    b, t, d = x.shape
    in_dim = ln_pre_w.shape[0]
    k_pool = in_dim // d
    t2 = (t // k_pool) * k_pool
    n = b * (t2 // k_pool)
    n_experts = router_w.shape[1]
    hid = sh_gate.shape[1]
    out_dim = sh_down.shape[1]

    xp = x[:, :t2, :].reshape(n, in_dim)
    full = lambda shape: pl.BlockSpec(shape, lambda e: (0,) * len(shape))
    anyspec = pl.BlockSpec(memory_space=pl.ANY)

    out = pl.pallas_call(
        functools.partial(_moe_kernel, n_experts=n_experts),
        grid=(n_experts,),
        in_specs=[
            full((n, in_dim)),
            full((1, in_dim)),
            full((in_dim, n_experts)),
            full((in_dim, hid)),
            full((in_dim, hid)),
            full((hid, out_dim)),
            anyspec,
            anyspec,
            anyspec,
            full((1, out_dim)),
            full((1, out_dim)),
        ],
        out_specs=full((n, out_dim)),
        out_shape=jax.ShapeDtypeStruct((n, out_dim), jnp.float32),
        scratch_shapes=[
            pltpu.VMEM((n, in_dim), jnp.bfloat16),
            pltpu.VMEM((n, 1), jnp.int32),
            pltpu.VMEM((n, 1), jnp.int32),
            pltpu.VMEM((n, 1), jnp.float32),
            pltpu.VMEM((n, out_dim), jnp.float32),
            pltpu.VMEM((2, in_dim, hid), jnp.float32),
            pltpu.VMEM((2, in_dim, hid), jnp.float32),
            pltpu.VMEM((2, hid, out_dim), jnp.float32),
            pltpu.SemaphoreType.DMA((2, 2)),
            pltpu.SemaphoreType.DMA((2, 2)),
            pltpu.SemaphoreType.DMA((2, 2)),
        ],
    )(xp, ln_pre_w.reshape(1, in_dim), router_w, sh_gate, sh_up, sh_down,
      eg, eu, ed, layer_scale.reshape(1, out_dim), ln_post_w.reshape(1, out_dim))
    return out.reshape(b, t2 // k_pool, out_dim)
